# baseline (speedup 1.0000x reference)
import functools

import jax
import jax.numpy as jnp
from jax import lax
from jax.experimental import pallas as pl
from jax.experimental.pallas import tpu as pltpu
from jax.experimental.pallas import tpu_sc as plsc

B = 16384
D = 64
L = 16
NC = 2
NS = 16
NW = NC * NS
BPW = B // NW
CHUNK = 128
NCHUNK = BPW // CHUNK
HALF = BPW // 2

_mesh = plsc.VectorSubcoreMesh(core_axis_name="c", subcore_axis_name="s")


@functools.partial(
    pl.kernel,
    mesh=_mesh,
    compiler_params=pltpu.CompilerParams(
        needs_layout_passes=False, use_tc_tiling_on_sc=True),
    out_type=jax.ShapeDtypeStruct((B,), jnp.float32),
    scratch_types=[
        pltpu.VMEM((NCHUNK, CHUNK), jnp.int32),
        pltpu.VMEM((NCHUNK, CHUNK), jnp.int32),
        pltpu.VMEM((NCHUNK, CHUNK), jnp.int32),
        pltpu.VMEM((NCHUNK, CHUNK), jnp.int32),
        pltpu.VMEM((HALF, 2 * D), jnp.float32),
        pltpu.VMEM((HALF, 2 * D), jnp.float32),
        pltpu.VMEM((BPW,), jnp.float32),
        pltpu.VMEM((BPW,), jnp.float32),
        pltpu.VMEM((D,), jnp.float32),
        pltpu.VMEM((L,), jnp.float32),
        pltpu.VMEM((BPW + 2 * L,), jnp.float32),
        pltpu.SemaphoreType.DMA,
    ],
)
def _gmf_sc(uid_hbm, iid_hbm, utab2, itab2, w_hbm, bias_hbm, out_hbm,
            uidx, iidx, uidx2, iidx2, urows, irows, upar, ipar, wv, bv, outv,
            sem):
    wid = lax.axis_index("s") * NC + lax.axis_index("c")
    base = wid * BPW

    pltpu.sync_copy(uid_hbm.at[wid], uidx)
    pltpu.sync_copy(iid_hbm.at[wid], iidx)
    pltpu.sync_copy(w_hbm, wv)
    pltpu.sync_copy(bias_hbm, bv)

    for c in range(NCHUNK):
        for k in range(CHUNK // L):
            s = pl.ds(L * k, L)
            fs = pl.ds(c * CHUNK + L * k, L)
            uv = uidx[c, s]
            iv = iidx[c, s]
            uidx2[c, s] = lax.shift_right_logical(uv, 1)
            iidx2[c, s] = lax.shift_right_logical(iv, 1)
            upar[fs] = lax.convert_element_type(
                lax.bitwise_and(uv, 1), jnp.float32)
            ipar[fs] = lax.convert_element_type(
                lax.bitwise_and(iv, 1), jnp.float32)

    w_slices = [wv[pl.ds(L * j, L)] for j in range(D // L)]
    bias = bv[...]
    lane0 = lax.iota(jnp.int32, L) == 0

    for h in range(2):
        copies = []
        for cc in range(2):
            c = 2 * h + cc
            dst = pl.ds(cc * CHUNK, CHUNK)
            copies.append(pltpu.async_copy(
                utab2.at[uidx2.at[c]], urows.at[dst], sem))
            copies.append(pltpu.async_copy(
                itab2.at[iidx2.at[c]], irows.at[dst], sem))
        for cp in copies:
            cp.wait()

        def rowsum(r, carry, h=h):
            g = h * HALF + r
            gsplat = jnp.full((L,), g, jnp.int32)
            pu = plsc.load_gather(upar, [gsplat]) > 0.5
            pi = plsc.load_gather(ipar, [gsplat]) > 0.5
            acc = bias
            for j in range(D // L):
                u = jnp.where(pu, urows[r, pl.ds(D + L * j, L)],
                              urows[r, pl.ds(L * j, L)])
                it = jnp.where(pi, irows[r, pl.ds(D + L * j, L)],
                               irows[r, pl.ds(L * j, L)])
                acc = acc + u * it * w_slices[j]
            sv = jnp.full((L,), jnp.sum(acc), jnp.float32)
            plsc.store_compressed(outv.at[pl.ds(g, L)], sv, mask=lane0)
            return carry

        lax.fori_loop(0, HALF, rowsum, 0)

    pltpu.sync_copy(outv.at[pl.ds(0, BPW)], out_hbm.at[pl.ds(base, BPW)])


def kernel(user_ids, item_ids, user_table, item_table, W, b):
    uid = user_ids.astype(jnp.int32).reshape(NW, NCHUNK, CHUNK)
    iid = item_ids.astype(jnp.int32).reshape(NW, NCHUNK, CHUNK)
    utab2 = user_table.reshape(-1, 2 * D)
    itab2 = item_table.reshape(-1, 2 * D)
    w64 = W.reshape(D).astype(jnp.float32)
    bias = jnp.zeros((L,), dtype=jnp.float32).at[0].set(b[0])
    return _gmf_sc(uid, iid, utab2, itab2, w64, bias)
